# Initial kernel scaffold; baseline (speedup 1.0000x reference)
import jax
import jax.numpy as jnp
from jax.experimental import pallas as pl


def kernel(node_feats, positions, senders, receivers, W_sc, W_pre0, W_pre1, W_pre2, W_pre3, W_post0, W_post1, W_post2, W_post3):
    raise NotImplementedError("write your pallas kernel here")


# speedup vs baseline: 28.9143x; 28.9143x over previous
import functools

import jax
import jax.numpy as jnp
from jax.experimental import pallas as pl

_AVG_NUM_NEIGHBORS = 16.0
_CHUNK = 1600
_NW = 256


def _edge_kernel(sx_ref, vec_ref, r_ref, wcat_ref, out_ref, *, n_nodes, m_dims):
    m0, m1, m2, m3 = m_dims
    i = pl.program_id(0)

    @pl.when(i == 0)
    def _zero():
        out_ref[...] = jnp.zeros_like(out_ref)

    sx = sx_ref[...]
    vec = vec_ref[...]
    r = r_ref[0, 0, :]

    u = jax.lax.dot(sx, wcat_ref[...], preferred_element_type=jnp.float32)
    u0 = u[:, :m0]
    u1 = u[:, m0:m0 + m1]
    u2 = u[:, m0 + m1:m0 + m1 + m2]
    u3 = u[:, m0 + m1 + m2:]

    norm = jnp.sqrt(jnp.sum(vec * vec, axis=1, keepdims=True))
    v = vec / (norm + 1e-9)
    x = v[:, 0:1]
    y = v[:, 1:2]
    z = v[:, 2:3]
    s3 = 3.0 ** 0.5
    s5 = 5.0 ** 0.5
    s7 = 7.0 ** 0.5
    s15 = 15.0 ** 0.5
    s42 = 42.0 ** 0.5
    s70 = 70.0 ** 0.5
    s105 = 105.0 ** 0.5
    sh1 = s3 * jnp.concatenate([x, y, z], axis=1)
    sh2 = jnp.concatenate([
        s15 * x * y,
        s15 * y * z,
        0.5 * s5 * (3.0 * z * z - 1.0),
        s15 * x * z,
        0.5 * s15 * (x * x - y * y),
    ], axis=1)
    sh3 = jnp.concatenate([
        0.25 * s70 * y * (3.0 * x * x - y * y),
        s105 * x * y * z,
        0.25 * s42 * y * (5.0 * z * z - 1.0),
        0.5 * s7 * z * (5.0 * z * z - 3.0),
        0.25 * s42 * x * (5.0 * z * z - 1.0),
        0.5 * s105 * z * (x * x - y * y),
        0.25 * s70 * x * (x * x - y * y),
    ], axis=1)

    def outer(u_l, sh_l, m_l, l_w):
        d_l = m_l * l_w
        rep = (jax.lax.broadcasted_iota(jnp.int32, (m_l, d_l), 1) // l_w ==
               jax.lax.broadcasted_iota(jnp.int32, (m_l, d_l), 0)
               ).astype(jnp.float32)
        til = (jax.lax.broadcasted_iota(jnp.int32, (l_w, d_l), 1) % l_w ==
               jax.lax.broadcasted_iota(jnp.int32, (l_w, d_l), 0)
               ).astype(jnp.float32)
        return (jax.lax.dot(u_l, rep, preferred_element_type=jnp.float32) *
                jax.lax.dot(sh_l, til, preferred_element_type=jnp.float32))

    f1 = outer(u1, sh1, m1, 3)
    f2 = outer(u2, sh2, m2, 5)
    f3 = outer(u3, sh3, m3, 7)
    feats = jnp.concatenate([u0, f1, f2, f3], axis=1)

    r2 = r.reshape(-1, 1)
    col = jax.lax.broadcasted_iota(jnp.int32, (r2.shape[0], _NW), 1)

    def cond(base):
        return base < n_nodes

    def body(base):
        base_al = jnp.maximum(jnp.minimum((base // 8) * 8, n_nodes - _NW), 0)
        base_al = pl.multiple_of(base_al, 8)
        oh = (r2 - base_al == col).astype(jnp.float32)
        contrib = jax.lax.dot_general(
            oh, feats, (((0,), (0,)), ((), ())),
            preferred_element_type=jnp.float32)
        out_ref[pl.ds(base_al, _NW), :] += contrib
        nxt = jnp.min(jnp.where(r2 >= base_al + _NW, r2, n_nodes))
        return nxt

    jax.lax.while_loop(cond, body, r2[0, 0])


def _node_kernel(nf_ref, agg_ref, wsc_ref, wp0_ref, bd_ref, out_ref, *, m0):
    inv = 1.0 / (_AVG_NUM_NEIGHBORS ** 0.5)
    nf = nf_ref[...]
    agg = agg_ref[...]
    a0 = agg[:, :m0]
    rest = agg[:, m0:]
    h0 = jax.lax.dot(jax.nn.gelu(a0 * inv), wp0_ref[...],
                     preferred_element_type=jnp.float32)
    out0 = jax.lax.dot(nf, wsc_ref[...],
                       preferred_element_type=jnp.float32) + h0
    out_rest = jax.lax.dot(rest, bd_ref[...],
                           preferred_element_type=jnp.float32)
    out_ref[...] = jnp.concatenate([out0, out_rest], axis=1)


@jax.jit
def kernel(node_feats, positions, senders, receivers, W_sc, W_pre0, W_pre1,
           W_pre2, W_pre3, W_post0, W_post1, W_post2, W_post3):
    n, c = node_feats.shape
    e = senders.shape[0]
    m0, m1, m2, m3 = (W_pre0.shape[1], W_pre1.shape[1],
                      W_pre2.shape[1], W_pre3.shape[1])
    d_rest = m1 * 3 + m2 * 5 + m3 * 7
    d = m0 + d_rest

    perm = jnp.argsort(receivers)
    r_s = receivers[perm].astype(jnp.int32)
    s_s = senders[perm]
    sx = node_feats[s_s]
    vec = positions[r_s] - positions[s_s]

    e_pad = ((e + _CHUNK - 1) // _CHUNK) * _CHUNK
    if e_pad != e:
        sx = jnp.pad(sx, ((0, e_pad - e), (0, 0)))
        vec = jnp.pad(vec, ((0, e_pad - e), (0, 0)))
        r_s = jnp.pad(r_s, (0, e_pad - e), constant_values=n - 1)
    nch = e_pad // _CHUNK
    r3 = r_s.reshape(nch, 1, _CHUNK)

    w_cat = jnp.concatenate([W_pre0, W_pre1, W_pre2, W_pre3], axis=1)

    agg = pl.pallas_call(
        functools.partial(_edge_kernel, n_nodes=n, m_dims=(m0, m1, m2, m3)),
        grid=(nch,),
        in_specs=[
            pl.BlockSpec((_CHUNK, c), lambda i: (i, 0)),
            pl.BlockSpec((_CHUNK, 3), lambda i: (i, 0)),
            pl.BlockSpec((1, 1, _CHUNK), lambda i: (i, 0, 0)),
            pl.BlockSpec((c, m0 + m1 + m2 + m3), lambda i: (0, 0)),
        ],
        out_specs=pl.BlockSpec((n, d), lambda i: (0, 0)),
        out_shape=jax.ShapeDtypeStruct((n, d), jnp.float32),
    )(sx, vec, r3, w_cat)

    inv = 1.0 / (_AVG_NUM_NEIGHBORS ** 0.5)
    bd = jax.scipy.linalg.block_diag(
        jnp.kron(W_post1, jnp.eye(3, dtype=jnp.float32)),
        jnp.kron(W_post2, jnp.eye(5, dtype=jnp.float32)),
        jnp.kron(W_post3, jnp.eye(7, dtype=jnp.float32))) * inv

    nb = 1000 if n % 1000 == 0 else n
    out = pl.pallas_call(
        functools.partial(_node_kernel, m0=m0),
        grid=(n // nb,),
        in_specs=[
            pl.BlockSpec((nb, c), lambda i: (i, 0)),
            pl.BlockSpec((nb, d), lambda i: (i, 0)),
            pl.BlockSpec((c, m0), lambda i: (0, 0)),
            pl.BlockSpec((m0, m0), lambda i: (0, 0)),
            pl.BlockSpec((d_rest, d_rest), lambda i: (0, 0)),
        ],
        out_specs=pl.BlockSpec((nb, d), lambda i: (0, 0)),
        out_shape=jax.ShapeDtypeStruct((n, d), jnp.float32),
    )(node_feats, agg, W_sc, W_post0, bd)
    return out
